# baseline (speedup 1.0000x reference)
import functools

import jax
import jax.numpy as jnp
from jax import lax
from jax.experimental import pallas as pl
from jax.experimental.pallas import tpu as pltpu
from jax.experimental.pallas import tpu_sc as plsc

B = 16384
F = 26
VOC = 100000
K = 64
NF = 13

NC = 2
NS = 16
NW = NC * NS
ROWS_PER_W = B // NW
CHUNK = 32
NCHUNK = ROWS_PER_W // CHUNK
GC = CHUNK * F


def _sc_gather_reduce(vc, ec_gran, idxt_flat, idx8_flat):
    mesh = plsc.VectorSubcoreMesh(core_axis_name="c", subcore_axis_name="s")

    @functools.partial(
        pl.kernel,
        mesh=mesh,
        compiler_params=pltpu.CompilerParams(use_tc_tiling_on_sc=False),
        out_type=(
            jax.ShapeDtypeStruct((B, K), jnp.float32),
            jax.ShapeDtypeStruct((B, 16), jnp.float32),
            jax.ShapeDtypeStruct((B * F, 8), jnp.float32),
        ),
        scratch_types=[
            pltpu.VMEM((GC,), jnp.int32),
            pltpu.VMEM((GC,), jnp.int32),
            pltpu.VMEM((GC, K), jnp.float32),
            pltpu.VMEM((GC, 8), jnp.float32),
            pltpu.VMEM((CHUNK, K), jnp.float32),
            pltpu.VMEM((CHUNK, 16), jnp.float32),
            pltpu.SemaphoreType.DMA,
            pltpu.SemaphoreType.DMA,
        ],
    )
    def k(vc_hbm, ec_hbm, idxt_hbm, idx8_hbm, scat_hbm, qpart_hbm, ecg_hbm,
          idxt_v, idx8_v, rows_v, ecg_v, scat_v, qpart_v, sem1, sem2):
        wid = lax.axis_index("s") * NC + lax.axis_index("c")
        wbase = wid * ROWS_PER_W

        def chunk_body(c, carry):
            rbase = wbase + c * CHUNK
            pltpu.sync_copy(idxt_hbm.at[pl.ds(rbase * F, GC)], idxt_v)
            pltpu.sync_copy(idx8_hbm.at[pl.ds(rbase * F, GC)], idx8_v)
            cp2 = pltpu.async_copy(ec_hbm.at[idx8_v], ecg_v, sem2)
            cps = []
            for f in range(F):
                cps.append(pltpu.async_copy(
                    vc_hbm.at[f].at[idxt_v.at[pl.ds(f * CHUNK, CHUNK)]],
                    rows_v.at[pl.ds(f * CHUNK, CHUNK)], sem1))
            for cp in cps:
                cp.wait()
            cp2.wait()

            def row_body(r, carry2):
                acc = [None] * 4
                qac = [None] * 4
                for f in range(F):
                    for j in range(4):
                        v = rows_v[f * CHUNK + r, pl.ds(j * 16, 16)]
                        if f == 0:
                            acc[j] = v
                            qac[j] = v * v
                        else:
                            acc[j] = acc[j] + v
                            qac[j] = qac[j] + v * v
                for j in range(4):
                    scat_v[r, pl.ds(j * 16, 16)] = acc[j]
                qpart_v[r, :] = (qac[0] + qac[1]) + (qac[2] + qac[3])
                return carry2

            lax.fori_loop(0, CHUNK, row_body, 0)
            pltpu.sync_copy(scat_v, scat_hbm.at[pl.ds(rbase, CHUNK)])
            pltpu.sync_copy(qpart_v, qpart_hbm.at[pl.ds(rbase, CHUNK)])
            pltpu.sync_copy(ecg_v, ecg_hbm.at[pl.ds(rbase * F, GC)])
            return carry

        lax.fori_loop(0, NCHUNK, chunk_body, 0)

    return k(vc, ec_gran, idxt_flat, idx8_flat)


def _tc_combine_body(xn_ref, wn_ref, vn_ref, scat_ref, qpart_ref, ecg_ref,
                     msk_ref, bias_ref, out_ref):
    xn = xn_ref[...]
    vn = vn_ref[...]
    s = jnp.dot(xn, vn, preferred_element_type=jnp.float32) + scat_ref[...]
    ss = jnp.sum(s * s, axis=1, keepdims=True)
    qc = jnp.sum(qpart_ref[...], axis=1, keepdims=True)
    ecsel = ecg_ref[...] * msk_ref[...].astype(jnp.float32)
    lc = jnp.sum(ecsel, axis=1, keepdims=True)
    vnsq = jnp.sum(vn * vn, axis=1, keepdims=True)
    qn = jnp.dot(xn * xn, vnsq, preferred_element_type=jnp.float32)
    lin = jnp.dot(xn, wn_ref[...], preferred_element_type=jnp.float32)
    out_ref[...] = lin + lc + 0.5 * (ss - (qn + qc)) + bias_ref[0, 0]


def _tc_combine(xn, Wn, Vn, scat, qpart, ecg, msk, bias):
    blk = 2048
    grid = B // blk
    return pl.pallas_call(
        _tc_combine_body,
        grid=(grid,),
        in_specs=[
            pl.BlockSpec((blk, NF), lambda i: (i, 0)),
            pl.BlockSpec((NF, 1), lambda i: (0, 0)),
            pl.BlockSpec((NF, K), lambda i: (0, 0)),
            pl.BlockSpec((blk, K), lambda i: (i, 0)),
            pl.BlockSpec((blk, 16), lambda i: (i, 0)),
            pl.BlockSpec((blk, F * 8), lambda i: (i, 0)),
            pl.BlockSpec((blk, F * 8), lambda i: (i, 0)),
            pl.BlockSpec((1, 1), lambda i: (0, 0)),
        ],
        out_specs=pl.BlockSpec((blk, 1), lambda i: (i, 0)),
        out_shape=jax.ShapeDtypeStruct((B, 1), jnp.float32),
    )(xn, Wn[:, None], Vn, scat, qpart, ecg, msk, bias[:, None])


def kernel(xn, xc, Wn, Vn, Ec, Vc, bias):
    idxt = jnp.transpose(
        xc.reshape(B // CHUNK, CHUNK, F), (0, 2, 1)).reshape(-1)
    idx8 = ((xc >> 3)
            + (jnp.arange(F, dtype=jnp.int32) * (VOC // 8))[None, :]).reshape(-1)
    ec_gran = Ec.reshape(F * VOC // 8, 8)
    scat, qpart, ecg = _sc_gather_reduce(Vc, ec_gran, idxt, idx8)
    msk = (
        (xc & 7)[:, :, None] == jnp.arange(8, dtype=jnp.int32)[None, None, :]
    ).reshape(B, F * 8).astype(jnp.bfloat16)
    return _tc_combine(xn, Wn, Vn, scat, qpart, ecg.reshape(B, F * 8), msk,
                       bias)


# speedup vs baseline: 1.4926x; 1.4926x over previous
import functools

import jax
import jax.numpy as jnp
from jax import lax
from jax.experimental import layout as jlayout
from jax.experimental import pallas as pl
from jax.experimental.pallas import tpu as pltpu
from jax.experimental.pallas import tpu_sc as plsc

B = 16384
F = 26
VOC = 100000
K = 64
NF = 13

NC = 2
NS = 16
NW = NC * NS
ROWS_PER_W = B // NW
CHUNK = 32
NCHUNK = ROWS_PER_W // CHUNK
GC = CHUNK * F


def _sc_gather_reduce(vc, ec_gran, idxt_flat, idx8_flat):
    mesh = plsc.VectorSubcoreMesh(core_axis_name="c", subcore_axis_name="s")

    @functools.partial(
        pl.kernel,
        mesh=mesh,
        compiler_params=pltpu.CompilerParams(use_tc_tiling_on_sc=False),
        out_type=(
            jax.ShapeDtypeStruct((B, K), jnp.float32),
            jax.ShapeDtypeStruct((B, 16), jnp.float32),
            jax.ShapeDtypeStruct((B * F, 8), jnp.float32),
        ),
        scratch_types=[
            pltpu.VMEM((GC,), jnp.int32),
            pltpu.VMEM((GC,), jnp.int32),
            pltpu.VMEM((GC, K), jnp.float32),
            pltpu.VMEM((GC, 8), jnp.float32),
            pltpu.VMEM((CHUNK, K), jnp.float32),
            pltpu.VMEM((CHUNK, 16), jnp.float32),
            pltpu.SemaphoreType.DMA,
            pltpu.SemaphoreType.DMA,
        ],
    )
    def k(vc_hbm, ec_hbm, idxt_hbm, idx8_hbm, scat_hbm, qpart_hbm, ecg_hbm,
          idxt_v, idx8_v, rows_v, ecg_v, scat_v, qpart_v, sem1, sem2):
        wid = lax.axis_index("s") * NC + lax.axis_index("c")
        wbase = wid * ROWS_PER_W

        def chunk_body(c, carry):
            rbase = wbase + c * CHUNK
            pltpu.sync_copy(idxt_hbm.at[pl.ds(rbase * F, GC)], idxt_v)
            pltpu.sync_copy(idx8_hbm.at[pl.ds(rbase * F, GC)], idx8_v)
            cp2 = pltpu.async_copy(ec_hbm.at[idx8_v], ecg_v, sem2)
            cps = []
            for f in range(F):
                cps.append(pltpu.async_copy(
                    vc_hbm.at[f].at[idxt_v.at[pl.ds(f * CHUNK, CHUNK)]],
                    rows_v.at[pl.ds(f * CHUNK, CHUNK)], sem1))
            for cp in cps:
                cp.wait()
            cp2.wait()

            def row_body(r, carry2):
                acc = [None] * 4
                qac = [None] * 4
                for f in range(F):
                    for j in range(4):
                        v = rows_v[f * CHUNK + r, pl.ds(j * 16, 16)]
                        if f == 0:
                            acc[j] = v
                            qac[j] = v * v
                        else:
                            acc[j] = acc[j] + v
                            qac[j] = qac[j] + v * v
                for j in range(4):
                    scat_v[r, pl.ds(j * 16, 16)] = acc[j]
                qpart_v[r, :] = (qac[0] + qac[1]) + (qac[2] + qac[3])
                return carry2

            lax.fori_loop(0, CHUNK, row_body, 0)
            pltpu.sync_copy(scat_v, scat_hbm.at[pl.ds(rbase, CHUNK)])
            pltpu.sync_copy(qpart_v, qpart_hbm.at[pl.ds(rbase, CHUNK)])
            pltpu.sync_copy(ecg_v, ecg_hbm.at[pl.ds(rbase * F, GC)])
            return carry

        lax.fori_loop(0, NCHUNK, chunk_body, 0)

    return k(vc, ec_gran, idxt_flat, idx8_flat)


def _tc_combine_body(xn_ref, wn_ref, vn_ref, scat_ref, qpart_ref, ecg_ref,
                     msk_ref, bias_ref, out_ref):
    xn = xn_ref[...]
    vn = vn_ref[...]
    s = jnp.dot(xn, vn, preferred_element_type=jnp.float32) + scat_ref[...]
    ss = jnp.sum(s * s, axis=1, keepdims=True)
    qc = jnp.sum(qpart_ref[...], axis=1, keepdims=True)
    ecsel = ecg_ref[...] * msk_ref[...].astype(jnp.float32)
    lc = jnp.sum(ecsel, axis=1, keepdims=True)
    vnsq = jnp.sum(vn * vn, axis=1, keepdims=True)
    qn = jnp.dot(xn * xn, vnsq, preferred_element_type=jnp.float32)
    lin = jnp.dot(xn, wn_ref[...], preferred_element_type=jnp.float32)
    out_ref[...] = lin + lc + 0.5 * (ss - (qn + qc)) + bias_ref[0, 0]


def _tc_combine(xn, Wn, Vn, scat, qpart, ecg, msk, bias):
    blk = 2048
    grid = B // blk
    return pl.pallas_call(
        _tc_combine_body,
        grid=(grid,),
        in_specs=[
            pl.BlockSpec((blk, NF), lambda i: (i, 0)),
            pl.BlockSpec((NF, 1), lambda i: (0, 0)),
            pl.BlockSpec((NF, K), lambda i: (0, 0)),
            pl.BlockSpec((blk, K), lambda i: (i, 0)),
            pl.BlockSpec((blk, 16), lambda i: (i, 0)),
            pl.BlockSpec((blk, F * 8), lambda i: (i, 0)),
            pl.BlockSpec((blk, F * 8), lambda i: (i, 0)),
            pl.BlockSpec((1, 1), lambda i: (0, 0)),
        ],
        out_specs=pl.BlockSpec((blk, 1), lambda i: (i, 0)),
        out_shape=jax.ShapeDtypeStruct((B, 1), jnp.float32),
    )(xn, Wn[:, None], Vn, scat, qpart, ecg, msk, bias[:, None])


def kernel(xn, xc, Wn, Vn, Ec, Vc, bias):
    idxt = jnp.transpose(
        xc.reshape(B // CHUNK, CHUNK, F), (0, 2, 1)).reshape(-1)
    idx8 = ((xc >> 3)
            + (jnp.arange(F, dtype=jnp.int32) * (VOC // 8))[None, :]).reshape(-1)
    ec_gran = Ec.reshape(F * VOC // 8, 8)
    vc_lin = jlayout.with_layout_constraint(
        Vc, jlayout.Layout((0, 1, 2), tiling=()))
    scat, qpart, ecg = _sc_gather_reduce(vc_lin, ec_gran, idxt, idx8)
    msk = (
        (xc & 7)[:, :, None] == jnp.arange(8, dtype=jnp.int32)[None, None, :]
    ).reshape(B, F * 8).astype(jnp.bfloat16)
    return _tc_combine(xn, Wn, Vn, scat, qpart, ecg.reshape(B, F * 8), msk,
                       bias)
